# baseline (speedup 1.0000x reference)
import functools

import jax
import jax.numpy as jnp
from jax import lax
from jax.experimental import pallas as pl
from jax.experimental.pallas import tpu as pltpu
from jax.experimental.pallas import tpu_sc as plsc

NUM_CLASSES = 256
D = 512
N = 32768
EPS = 1e-08

NC = 2
NS = 16
NW = NC * NS
NQ = 4
NSTRIPE = NW // NQ
DQ = D // NQ
ROWS_PER_W = N // NSTRIPE
CHUNK = 128
NCHUNKS = ROWS_PER_W // CHUNK
LBLK = 4096

_mesh = plsc.VectorSubcoreMesh(core_axis_name="c", subcore_axis_name="s")


@functools.partial(
    pl.kernel,
    out_type=jax.ShapeDtypeStruct((NW * 2 * NUM_CLASSES, DQ), jnp.float32),
    mesh=_mesh,
    scratch_types=[
        pltpu.VMEM((CHUNK, DQ), jnp.float32),
        pltpu.VMEM((CHUNK,), jnp.int32),
        pltpu.VMEM((NUM_CLASSES, DQ), jnp.float32),
        pltpu.VMEM((NUM_CLASSES, DQ), jnp.float32),
    ],
)
def _sc_segment_sum(feat_hbm, lab_hbm, psums_hbm, rows_v, lab_v, acc0, acc1):
    c = lax.axis_index("c")
    s = lax.axis_index("s")
    q = c * 2 + s // 8
    stripe = s % 8
    wid = q * NSTRIPE + stripe
    base = stripe * ROWS_PER_W
    col0 = q * DQ

    zeros16 = jnp.zeros((16,), jnp.float32)

    def _zacc(i, _):
        for j in range(DQ // 16):
            acc0[i, pl.ds(j * 16, 16)] = zeros16
            acc1[i, pl.ds(j * 16, 16)] = zeros16
        return 0
    lax.fori_loop(0, NUM_CLASSES, _zacc, 0)

    def _chunk(k, _):
        r0 = base + k * CHUNK
        pltpu.sync_copy(feat_hbm.at[pl.ds(r0, CHUNK), pl.ds(col0, DQ)],
                        rows_v)
        pltpu.sync_copy(lab_hbm.at[pl.ds(r0, CHUNK)], lab_v)

        def _pair(p, _):
            ia = p * 32
            ib = ia + 16
            laba = lab_v[pl.ds(ia, 16)]
            labb = lab_v[pl.ds(ib, 16)]
            for l in range(16):
                la = laba[l]
                lb = labb[l]
                for j in range(DQ // 16):
                    plsc.addupdate(acc0.at[la, pl.ds(j * 16, 16)],
                                   rows_v[ia + l, pl.ds(j * 16, 16)])
                    plsc.addupdate(acc1.at[lb, pl.ds(j * 16, 16)],
                                   rows_v[ib + l, pl.ds(j * 16, 16)])
            return 0
        lax.fori_loop(0, CHUNK // 32, _pair, 0)
        return 0
    lax.fori_loop(0, NCHUNKS, _chunk, 0)

    out0 = pl.ds((wid * 2) * NUM_CLASSES, NUM_CLASSES)
    out1 = pl.ds((wid * 2 + 1) * NUM_CLASSES, NUM_CLASSES)
    pltpu.sync_copy(acc0, psums_hbm.at[out0])
    pltpu.sync_copy(acc1, psums_hbm.at[out1])


def _tc_finish(psums_ref, lab_ref, out_ref):
    psums = psums_ref[...]

    def _qsum(q):
        acc = None
        for st in range(NSTRIPE):
            for bank in range(2):
                i = ((q * NSTRIPE + st) * 2 + bank) * NUM_CLASSES
                blk = psums[i:i + NUM_CLASSES]
                acc = blk if acc is None else acc + blk
        return acc

    sums = [_qsum(q) for q in range(NQ)]

    cls = lax.broadcasted_iota(jnp.int32, (NUM_CLASSES, LBLK), 0)
    counts = jnp.zeros((NUM_CLASSES,), jnp.float32)
    for b in range(N // LBLK):
        blk = lab_ref[pl.ds(b, 1), :]
        eq = (blk == cls).astype(jnp.float32)
        counts = counts + jnp.sum(eq, axis=1)

    present = counts > 0.0
    safe = jnp.maximum(counts, 1.0)
    dims = (((1,), (1,)), ((), ()))
    norms = jnp.zeros((NUM_CLASSES,), jnp.float32)
    gram = jnp.zeros((NUM_CLASSES, NUM_CLASSES), jnp.float32)
    for q in range(NQ):
        cent = jnp.where(present[:, None], sums[q] / safe[:, None], 0.0)
        norms = norms + jnp.sum(cent * cent, axis=1)
        gram = gram + lax.dot_general(cent, cent, dims,
                                      preferred_element_type=jnp.float32,
                                      precision=lax.Precision.HIGHEST)
    dist_sq = jnp.maximum(norms[:, None] + norms[None, :] - 2.0 * gram, 0.0)
    ii = lax.broadcasted_iota(jnp.int32, (NUM_CLASSES, NUM_CLASSES), 0)
    jj = lax.broadcasted_iota(jnp.int32, (NUM_CLASSES, NUM_CLASSES), 1)
    valid = (ii < jj) & present[:, None] & present[None, :]
    safe_sq = jnp.where(valid, dist_sq, 1.0)
    distance = jnp.sqrt(safe_sq) / 16.0
    terms = jnp.where(valid, jnp.exp(-(distance + EPS)), 0.0)
    out_ref[...] = jnp.sum(terms).reshape(1, 1)


_finish = pl.pallas_call(
    _tc_finish,
    out_shape=jax.ShapeDtypeStruct((1, 1), jnp.float32),
)


def kernel(features, labels):
    labels = labels.astype(jnp.int32)
    psums = _sc_segment_sum(features, labels)
    loss = _finish(psums, labels.reshape(N // LBLK, LBLK))
    return loss.reshape(())


# speedup vs baseline: 2.6974x; 2.6974x over previous
import functools

import jax
import jax.numpy as jnp
from jax import lax
from jax.experimental import pallas as pl
from jax.experimental.pallas import tpu as pltpu
from jax.experimental.pallas import tpu_sc as plsc

NUM_CLASSES = 256
D = 512
N = 32768
EPS = 1e-08

NC = 2
NS = 16
NW = NC * NS
NQ = 4
NSTRIPE = NW // NQ
DQ = D // NQ
ROWS_PER_W = N // NSTRIPE
CHUNK = 128
NCHUNKS = ROWS_PER_W // CHUNK
LBLK = 4096

_mesh = plsc.VectorSubcoreMesh(core_axis_name="c", subcore_axis_name="s")


@functools.partial(
    pl.kernel,
    out_type=jax.ShapeDtypeStruct((NW * 2 * NUM_CLASSES, DQ), jnp.float32),
    mesh=_mesh,
    scratch_types=[
        pltpu.VMEM((CHUNK, DQ), jnp.float32),
        pltpu.VMEM((CHUNK, DQ), jnp.float32),
        pltpu.VMEM((CHUNK,), jnp.int32),
        pltpu.VMEM((CHUNK,), jnp.int32),
        pltpu.VMEM((NUM_CLASSES, DQ), jnp.float32),
        pltpu.VMEM((NUM_CLASSES, DQ), jnp.float32),
        pltpu.SemaphoreType.DMA,
        pltpu.SemaphoreType.DMA,
        pltpu.SemaphoreType.DMA,
        pltpu.SemaphoreType.DMA,
    ],
)
def _sc_segment_sum(feat_hbm, lab_hbm, psums_hbm, rows_a, rows_b,
                    lab_a, lab_b, acc0, acc1, sra, srb, sla, slb):
    c = lax.axis_index("c")
    s = lax.axis_index("s")
    q = c * 2 + s // 8
    stripe = s % 8
    wid = q * NSTRIPE + stripe
    base = stripe * ROWS_PER_W
    col0 = q * DQ

    zeros16 = jnp.zeros((16,), jnp.float32)

    def _feat(k):
        return feat_hbm.at[pl.ds(base + k * CHUNK, CHUNK), pl.ds(col0, DQ)]

    def _lab(k):
        return lab_hbm.at[pl.ds(base + k * CHUNK, CHUNK)]

    pltpu.async_copy(_feat(0), rows_a, sra)
    pltpu.async_copy(_lab(0), lab_a, sla)

    def _zacc(i, _):
        for j in range(DQ // 16):
            acc0[i, pl.ds(j * 16, 16)] = zeros16
            acc1[i, pl.ds(j * 16, 16)] = zeros16
        return 0
    lax.fori_loop(0, NUM_CLASSES, _zacc, 0)

    def _process(rows_v, lab_v):
        def _pair(p, _):
            ia = p * 32
            ib = ia + 16
            laba = lab_v[pl.ds(ia, 16)]
            labb = lab_v[pl.ds(ib, 16)]
            def _loads(l):
                ra = [rows_v[ia + l, pl.ds(j * 16, 16)]
                      for j in range(DQ // 16)]
                rb = [rows_v[ib + l, pl.ds(j * 16, 16)]
                      for j in range(DQ // 16)]
                return ra, rb

            ra, rb = _loads(0)
            for l in range(16):
                if l < 15:
                    na, nb = _loads(l + 1)
                la = laba[l]
                lb = labb[l]
                for j in range(DQ // 16):
                    plsc.addupdate(acc0.at[la, pl.ds(j * 16, 16)], ra[j])
                    plsc.addupdate(acc1.at[lb, pl.ds(j * 16, 16)], rb[j])
                if l < 15:
                    ra, rb = na, nb
            return 0
        lax.fori_loop(0, CHUNK // 32, _pair, 0)

    def _loop2(t, _):
        k0 = 2 * t
        pltpu.async_copy(_feat(k0 + 1), rows_b, srb)
        pltpu.async_copy(_lab(k0 + 1), lab_b, slb)
        pltpu.make_async_copy(_feat(k0), rows_a, sra).wait()
        pltpu.make_async_copy(_lab(k0), lab_a, sla).wait()
        _process(rows_a, lab_a)

        @pl.when(t < NCHUNKS // 2 - 1)
        def _():
            pltpu.async_copy(_feat(k0 + 2), rows_a, sra)
            pltpu.async_copy(_lab(k0 + 2), lab_a, sla)

        pltpu.make_async_copy(_feat(k0 + 1), rows_b, srb).wait()
        pltpu.make_async_copy(_lab(k0 + 1), lab_b, slb).wait()
        _process(rows_b, lab_b)
        return 0
    lax.fori_loop(0, NCHUNKS // 2, _loop2, 0)

    out0 = pl.ds(((stripe * NQ + q) * 2) * NUM_CLASSES, NUM_CLASSES)
    out1 = pl.ds(((stripe * NQ + q) * 2 + 1) * NUM_CLASSES, NUM_CLASSES)
    pltpu.sync_copy(acc0, psums_hbm.at[out0])
    pltpu.sync_copy(acc1, psums_hbm.at[out1])


def _tc_finish(lab_ref, psums_ref, out_ref, cnt_v):
    step = pl.program_id(0)

    @pl.when(step == 0)
    def _():
        cls = lax.broadcasted_iota(jnp.int32, (NUM_CLASSES, LBLK), 0)
        counts = jnp.zeros((NUM_CLASSES, 1), jnp.float32)
        for b in range(N // LBLK):
            blk = lab_ref[pl.ds(b, 1), :]
            eq = (blk == cls).astype(jnp.float32)
            counts = counts + jnp.sum(eq, axis=1, keepdims=True)
        cnt_v[...] = counts

    @pl.when(step == 1)
    def _():
        psums = psums_ref[...]

        def _qsum(q):
            acc = None
            for st in range(NSTRIPE):
                for bank in range(2):
                    i = ((st * NQ + q) * 2 + bank) * NUM_CLASSES
                    blk = psums[i:i + NUM_CLASSES]
                    acc = blk if acc is None else acc + blk
            return acc

        sums = [_qsum(q) for q in range(NQ)]

        counts = cnt_v[...][:, 0]
        present = counts > 0.0
        safe = jnp.maximum(counts, 1.0)
        dims = (((1,), (1,)), ((), ()))
        norms = jnp.zeros((NUM_CLASSES,), jnp.float32)
        gram = jnp.zeros((NUM_CLASSES, NUM_CLASSES), jnp.float32)
        for q in range(NQ):
            cent = jnp.where(present[:, None], sums[q] / safe[:, None], 0.0)
            norms = norms + jnp.sum(cent * cent, axis=1)
            gram = gram + lax.dot_general(cent, cent, dims,
                                          preferred_element_type=jnp.float32,
                                          precision=lax.Precision.HIGHEST)
        dist_sq = jnp.maximum(
            norms[:, None] + norms[None, :] - 2.0 * gram, 0.0)
        ii = lax.broadcasted_iota(jnp.int32, (NUM_CLASSES, NUM_CLASSES), 0)
        jj = lax.broadcasted_iota(jnp.int32, (NUM_CLASSES, NUM_CLASSES), 1)
        valid = (ii < jj) & present[:, None] & present[None, :]
        safe_sq = jnp.where(valid, dist_sq, 1.0)
        distance = jnp.sqrt(safe_sq) / 16.0
        terms = jnp.where(valid, jnp.exp(-(distance + EPS)), 0.0)
        out_ref[...] = jnp.sum(terms).reshape(1, 1)


_finish = pl.pallas_call(
    _tc_finish,
    grid=(2,),
    in_specs=[
        pl.BlockSpec((N // LBLK, LBLK), lambda i: (0, 0)),
        pl.BlockSpec((NW * 2 * NUM_CLASSES, DQ), lambda i: (0, 0)),
    ],
    out_specs=pl.BlockSpec((1, 1), lambda i: (0, 0)),
    out_shape=jax.ShapeDtypeStruct((1, 1), jnp.float32),
    scratch_shapes=[pltpu.VMEM((NUM_CLASSES, 1), jnp.float32)],
)


def kernel(features, labels):
    labels = labels.astype(jnp.int32)
    psums = _sc_segment_sum(features, labels)
    loss = _finish(labels.reshape(N // LBLK, LBLK), psums)
    return loss.reshape(())


# speedup vs baseline: 2.8254x; 1.0475x over previous
import functools

import jax
import jax.numpy as jnp
from jax import lax
from jax.experimental import pallas as pl
from jax.experimental.pallas import tpu as pltpu
from jax.experimental.pallas import tpu_sc as plsc

NUM_CLASSES = 256
D = 512
N = 32768
EPS = 1e-08

NC = 2
NS = 16
NW = NC * NS
NQ = 4
NSTRIPE = NW // NQ
DQ = D // NQ
ROWS_PER_W = N // NSTRIPE
CHUNK = 128
NCHUNKS = ROWS_PER_W // CHUNK
LBLK = 4096

_mesh = plsc.VectorSubcoreMesh(core_axis_name="c", subcore_axis_name="s")


@functools.partial(
    pl.kernel,
    out_type=jax.ShapeDtypeStruct((NW * 2 * NUM_CLASSES, DQ), jnp.float32),
    mesh=_mesh,
    scratch_types=[
        pltpu.VMEM((CHUNK, DQ), jnp.float32),
        pltpu.VMEM((CHUNK, DQ), jnp.float32),
        pltpu.VMEM((CHUNK,), jnp.int32),
        pltpu.VMEM((CHUNK,), jnp.int32),
        pltpu.VMEM((NUM_CLASSES, DQ), jnp.float32),
        pltpu.VMEM((NUM_CLASSES, DQ), jnp.float32),
        pltpu.SemaphoreType.DMA,
        pltpu.SemaphoreType.DMA,
        pltpu.SemaphoreType.DMA,
        pltpu.SemaphoreType.DMA,
    ],
)
def _sc_segment_sum(feat_hbm, lab_hbm, psums_hbm, rows_a, rows_b,
                    lab_a, lab_b, acc0, acc1, sra, srb, sla, slb):
    c = lax.axis_index("c")
    s = lax.axis_index("s")
    q = c * 2 + s // 8
    stripe = s % 8
    wid = q * NSTRIPE + stripe
    base = stripe * ROWS_PER_W
    col0 = q * DQ

    zeros16 = jnp.zeros((16,), jnp.float32)

    def _feat(k):
        return feat_hbm.at[pl.ds(base + k * CHUNK, CHUNK), pl.ds(col0, DQ)]

    def _lab(k):
        return lab_hbm.at[pl.ds(base + k * CHUNK, CHUNK)]

    pltpu.async_copy(_feat(0), rows_a, sra)
    pltpu.async_copy(_lab(0), lab_a, sla)

    def _zacc(i, _):
        for j in range(DQ // 16):
            acc0[i, pl.ds(j * 16, 16)] = zeros16
            acc1[i, pl.ds(j * 16, 16)] = zeros16
        return 0
    lax.fori_loop(0, NUM_CLASSES, _zacc, 0)

    def _process(rows_v, lab_v):
        def _pair(p, _):
            ia = p * 32
            ib = ia + 16
            laba = lab_v[pl.ds(ia, 16)]
            labb = lab_v[pl.ds(ib, 16)]
            def _loads(l):
                ra = [rows_v[ia + l, pl.ds(j * 16, 16)]
                      for j in range(DQ // 16)]
                rb = [rows_v[ib + l, pl.ds(j * 16, 16)]
                      for j in range(DQ // 16)]
                return ra, rb

            ra, rb = _loads(0)
            for l in range(16):
                if l < 15:
                    na, nb = _loads(l + 1)
                la = laba[l]
                lb = labb[l]
                for j in range(DQ // 16):
                    plsc.addupdate(acc0.at[la, pl.ds(j * 16, 16)], ra[j])
                    plsc.addupdate(acc1.at[lb, pl.ds(j * 16, 16)], rb[j])
                if l < 15:
                    ra, rb = na, nb
            return 0
        lax.fori_loop(0, CHUNK // 32, _pair, 0)

    def _loop2(t, _):
        k0 = 2 * t
        pltpu.async_copy(_feat(k0 + 1), rows_b, srb)
        pltpu.async_copy(_lab(k0 + 1), lab_b, slb)
        pltpu.make_async_copy(_feat(k0), rows_a, sra).wait()
        pltpu.make_async_copy(_lab(k0), lab_a, sla).wait()
        _process(rows_a, lab_a)

        @pl.when(t < NCHUNKS // 2 - 1)
        def _():
            pltpu.async_copy(_feat(k0 + 2), rows_a, sra)
            pltpu.async_copy(_lab(k0 + 2), lab_a, sla)

        pltpu.make_async_copy(_feat(k0 + 1), rows_b, srb).wait()
        pltpu.make_async_copy(_lab(k0 + 1), lab_b, slb).wait()
        _process(rows_b, lab_b)
        return 0
    lax.fori_loop(0, NCHUNKS // 2, _loop2, 0)

    out0 = pl.ds(((stripe * NQ + q) * 2) * NUM_CLASSES, NUM_CLASSES)
    out1 = pl.ds(((stripe * NQ + q) * 2 + 1) * NUM_CLASSES, NUM_CLASSES)
    pltpu.sync_copy(acc0, psums_hbm.at[out0])
    pltpu.sync_copy(acc1, psums_hbm.at[out1])


def _tc_counts(lab_ref, out_ref):
    cls = lax.broadcasted_iota(jnp.int32, (NUM_CLASSES, LBLK), 0)
    counts = jnp.zeros((NUM_CLASSES, 1), jnp.float32)
    for b in range(N // LBLK):
        blk = lab_ref[pl.ds(b, 1), :]
        eq = (blk == cls).astype(jnp.float32)
        counts = counts + jnp.sum(eq, axis=1, keepdims=True)
    out_ref[...] = counts


_counts = pl.pallas_call(
    _tc_counts,
    out_shape=jax.ShapeDtypeStruct((NUM_CLASSES, 1), jnp.float32),
)


def _tc_finish(psums_ref, counts_ref, out_ref):
    psums = psums_ref[...]

    def _qsum(q):
        acc = None
        for st in range(NSTRIPE):
            for bank in range(2):
                i = ((st * NQ + q) * 2 + bank) * NUM_CLASSES
                blk = psums[i:i + NUM_CLASSES]
                acc = blk if acc is None else acc + blk
        return acc

    sums = [_qsum(q) for q in range(NQ)]

    counts = counts_ref[...][:, 0]
    present = counts > 0.0
    safe = jnp.maximum(counts, 1.0)
    dims = (((1,), (1,)), ((), ()))
    norms = jnp.zeros((NUM_CLASSES,), jnp.float32)
    gram = jnp.zeros((NUM_CLASSES, NUM_CLASSES), jnp.float32)
    for q in range(NQ):
        cent = jnp.where(present[:, None], sums[q] / safe[:, None], 0.0)
        norms = norms + jnp.sum(cent * cent, axis=1)
        gram = gram + lax.dot_general(cent, cent, dims,
                                      preferred_element_type=jnp.float32,
                                      precision=lax.Precision.HIGHEST)
    dist_sq = jnp.maximum(norms[:, None] + norms[None, :] - 2.0 * gram, 0.0)
    ii = lax.broadcasted_iota(jnp.int32, (NUM_CLASSES, NUM_CLASSES), 0)
    jj = lax.broadcasted_iota(jnp.int32, (NUM_CLASSES, NUM_CLASSES), 1)
    valid = (ii < jj) & present[:, None] & present[None, :]
    safe_sq = jnp.where(valid, dist_sq, 1.0)
    distance = jnp.sqrt(safe_sq) / 16.0
    terms = jnp.where(valid, jnp.exp(-(distance + EPS)), 0.0)
    out_ref[...] = jnp.sum(terms).reshape(1, 1)


_finish = pl.pallas_call(
    _tc_finish,
    out_shape=jax.ShapeDtypeStruct((1, 1), jnp.float32),
)


def kernel(features, labels):
    labels = labels.astype(jnp.int32)
    counts = _counts(labels.reshape(N // LBLK, LBLK))
    psums = _sc_segment_sum(features, labels)
    loss = _finish(psums, counts)
    return loss.reshape(())
